# baseline (speedup 1.0000x reference)
import functools

import jax
import jax.numpy as jnp
from jax import lax
from jax.experimental import pallas as pl
from jax.experimental.pallas import tpu as pltpu
from jax.experimental.pallas import tpu_sc as plsc

N = 10000
E = 320000
D = 128
H = 128
C = 16

NC = 2
NS = 16
NW = NC * NS
CHUNK = 112
NBUF = 3
SPS = 24
NSTAGE = 4
NCHUNK = SPS * NSTAGE
EPW = NCHUNK * CHUNK
EPAD = EPW * NW
DCHUNK = 128
EPWD = 10240
NPAD = 10240
SLAB = NPAD // NS

_MESH = plsc.VectorSubcoreMesh(core_axis_name="c", subcore_axis_name="s")

CPR = 1


def _idx(v, j):
    if CPR == 1:
        return v.at[j]
    return v.at[j // CPR, pl.ds((j % CPR) * CHUNK, CHUNK)]


def _unpack_indices(src_v, dst_v, nchunk, chunk):
    def body(j, carry):
        for k in range(chunk // 16):
            p = src_v[j, pl.ds(k * 16, 16)]
            src_v[j, pl.ds(k * 16, 16)] = lax.shift_right_logical(p, 14)
            dst_v[j, pl.ds(k * 16, 16)] = lax.bitwise_and(p, 16383)
        return carry

    lax.fori_loop(0, nchunk, body, 0)


def _deg_body(pk_hbm, zero_hbm, out_hbm, src_v, dst_v, ones_v,
              dego_s, degi_s):
    c = lax.axis_index("c")
    s = lax.axis_index("s")
    w = c * NS + s
    pltpu.sync_copy(zero_hbm.at[pl.ds(s * SLAB, SLAB)],
                    dego_s.at[pl.ds(s * SLAB, SLAB)])
    pltpu.sync_copy(zero_hbm.at[pl.ds(s * SLAB, SLAB)],
                    degi_s.at[pl.ds(s * SLAB, SLAB)])
    pltpu.sync_copy(pk_hbm.at[w], src_v)
    _unpack_indices(src_v, dst_v, EPWD // DCHUNK, DCHUNK)
    for k in range(DCHUNK // 16):
        ones_v[pl.ds(k * 16, 16)] = jnp.ones((16,), jnp.float32)
    plsc.subcore_barrier()

    def body(j, carry):
        pltpu.sync_copy(ones_v, dego_s.at[src_v.at[j]], add=True)
        pltpu.sync_copy(ones_v, degi_s.at[dst_v.at[j]], add=True)
        return carry

    lax.fori_loop(0, EPWD // DCHUNK, body, 0)
    plsc.subcore_barrier()
    pltpu.sync_copy(dego_s.at[pl.ds(s * SLAB, SLAB)],
                    out_hbm.at[c, 0, pl.ds(s * SLAB, SLAB)])
    pltpu.sync_copy(degi_s.at[pl.ds(s * SLAB, SLAB)],
                    out_hbm.at[c, 1, pl.ds(s * SLAB, SLAB)])


def _agg_body(hs_hbm, pk_hbm, zero_hbm, out_hbm, src_v, dst_v,
              b0, b1, b2, acc_s, g0, g1, g2, s0, s1, s2):
    c = lax.axis_index("c")
    s = lax.axis_index("s")
    w = c * NS + s
    bufs = [b0, b1, b2][:NBUF]
    gsem = [g0, g1, g2][:NBUF]
    ssem = [s0, s1, s2][:NBUF]
    pltpu.sync_copy(zero_hbm.at[pl.ds(s * SLAB, SLAB)],
                    acc_s.at[pl.ds(s * SLAB, SLAB)])
    plsc.subcore_barrier()

    def run_stage(h0):
        pltpu.sync_copy(pk_hbm.at[w, pl.ds(h0, SPS)], src_v)
        _unpack_indices(src_v, dst_v, SPS, CHUNK)

        def g_start(j, b):
            pltpu.async_copy(hs_hbm.at[_idx(src_v, j)], bufs[b], gsem[b])

        def g_wait(j, b):
            pltpu.make_async_copy(hs_hbm.at[_idx(src_v, j)], bufs[b],
                                  gsem[b]).wait()

        def s_start(j, b):
            pltpu.async_copy(bufs[b], acc_s.at[_idx(dst_v, j)], ssem[b],
                             add=True)

        def s_wait(j, b):
            pltpu.make_async_copy(bufs[b], acc_s.at[_idx(dst_v, j)],
                                  ssem[b]).wait()

        g_start(0, 0)

        def body(i, carry):
            for b in range(NBUF):
                j = NBUF * i + b
                g_wait(j, b)
                s_start(j, b)
                nb = (b + 1) % NBUF

                @pl.when(jnp.logical_and(j + 1 < SPS, j >= NBUF - 1))
                def _():
                    s_wait(j - (NBUF - 1), nb)

                @pl.when(j + 1 < SPS)
                def _():
                    g_start(j + 1, nb)

            return carry

        lax.fori_loop(0, SPS // NBUF, body, 0)
        for k in range(NBUF):
            s_wait(SPS - NBUF + k, (SPS - NBUF + k) % NBUF)

    for st in range(NSTAGE):
        run_stage(st * SPS)
    plsc.subcore_barrier()
    pltpu.sync_copy(acc_s.at[pl.ds(s * SLAB, SLAB)],
                    out_hbm.at[c, pl.ds(s * SLAB, SLAB)])


_deg_call = pl.kernel(
    _deg_body,
    out_type=jax.ShapeDtypeStruct((NC, 2, NPAD), jnp.float32),
    mesh=_MESH,
    scratch_types=[
        pltpu.VMEM((EPWD // DCHUNK, DCHUNK), jnp.int32),
        pltpu.VMEM((EPWD // DCHUNK, DCHUNK), jnp.int32),
        pltpu.VMEM((DCHUNK,), jnp.float32),
        pltpu.VMEM_SHARED((NPAD,), jnp.float32),
        pltpu.VMEM_SHARED((NPAD,), jnp.float32),
    ],
)

_agg_call = pl.kernel(
    _agg_body,
    out_type=jax.ShapeDtypeStruct((NC, NPAD, H), jnp.float32),
    mesh=_MESH,
    scratch_types=[
        pltpu.VMEM((SPS, CHUNK), jnp.int32),
        pltpu.VMEM((SPS, CHUNK), jnp.int32),
    ] + [pltpu.VMEM((CHUNK, H), jnp.float32)] * NBUF + [
        pltpu.VMEM_SHARED((NPAD, H), jnp.float32),
    ] + [pltpu.SemaphoreType.DMA] * (2 * NBUF),
)


def _norms(degp):
    normo = lax.rsqrt(jnp.maximum(degp[0, 0] + degp[1, 0], 1.0))
    normi = lax.rsqrt(jnp.maximum(degp[0, 1] + degp[1, 1], 1.0))
    return normo, normi


def _fc_body(x_ref, w_ref, b_ref, degp_ref, out_ref):
    normo, _ = _norms(degp_ref[...])
    h = jnp.dot(x_ref[...], w_ref[...], preferred_element_type=jnp.float32)
    out_ref[...] = (h + b_ref[...]) * normo


def _layer_a_body(p_ref, degp_ref, b_ref, out_ref):
    normo, normi = _norms(degp_ref[...])
    agg = (p_ref[0] + p_ref[1]) * normi
    h = jnp.maximum(agg + b_ref[...], 0.0)
    out_ref[...] = h * normo


def _layer_b_body(p_ref, degp_ref, w_ref, b_ref, h_ref, hs_ref):
    normo, normi = _norms(degp_ref[...])
    agg = (p_ref[0] + p_ref[1]) * normi
    h = jnp.dot(agg, w_ref[...], preferred_element_type=jnp.float32)
    h = jnp.maximum(h + b_ref[...], 0.0)
    h_ref[...] = h
    hs_ref[...] = h * normo


def _layer_c_body(p_ref, degp_ref, w_ref, b_ref, out_ref):
    _, normi = _norms(degp_ref[...])
    agg = (p_ref[0] + p_ref[1]) * normi
    h = jnp.dot(agg, w_ref[...], preferred_element_type=jnp.float32)
    out_ref[...] = h + b_ref[...]


_GRID = (N + 127) // 128

_degp_spec = pl.BlockSpec((NC, 2, 128, 1), lambda i: (0, 0, i, 0))
_row_spec = pl.BlockSpec((128, H), lambda i: (i, 0))
_p_spec = pl.BlockSpec((NC, 128, H), lambda i: (0, i, 0))


def _mk_fc():
    return pl.pallas_call(
        _fc_body,
        grid=(_GRID,),
        in_specs=[_row_spec,
                  pl.BlockSpec((D, H), lambda i: (0, 0)),
                  pl.BlockSpec((1, H), lambda i: (0, 0)),
                  _degp_spec],
        out_specs=_row_spec,
        out_shape=jax.ShapeDtypeStruct((N, H), jnp.float32),
    )


def _mk_layer_a():
    return pl.pallas_call(
        _layer_a_body,
        grid=(_GRID,),
        in_specs=[_p_spec, _degp_spec,
                  pl.BlockSpec((1, H), lambda i: (0, 0))],
        out_specs=_row_spec,
        out_shape=jax.ShapeDtypeStruct((N, H), jnp.float32),
    )


def _mk_layer_b():
    return pl.pallas_call(
        _layer_b_body,
        grid=(_GRID,),
        in_specs=[_p_spec, _degp_spec,
                  pl.BlockSpec((H, H), lambda i: (0, 0)),
                  pl.BlockSpec((1, H), lambda i: (0, 0))],
        out_specs=[_row_spec, _row_spec],
        out_shape=[jax.ShapeDtypeStruct((N, H), jnp.float32),
                   jax.ShapeDtypeStruct((N, H), jnp.float32)],
    )


def _mk_layer_c():
    return pl.pallas_call(
        _layer_c_body,
        grid=(_GRID,),
        in_specs=[_p_spec, _degp_spec,
                  pl.BlockSpec((H, C), lambda i: (0, 0)),
                  pl.BlockSpec((1, C), lambda i: (0, 0))],
        out_specs=pl.BlockSpec((128, C), lambda i: (i, 0)),
        out_shape=jax.ShapeDtypeStruct((N, C), jnp.float32),
    )


def kernel(features_list, edge_index, e_feat, W_fc, b_fc, b0, W1, b1, W2, b2):
    src = edge_index[0]
    dst = edge_index[1]
    def padded(epw, trash_src):
        padw = epw - E // NW
        tr = jnp.arange(padw, dtype=jnp.int32)
        t_dst = jnp.broadcast_to(N + tr % (NPAD - N), (NW, padw))
        t_src = t_dst if trash_src else jnp.broadcast_to(tr % N, (NW, padw))
        s_p = jnp.concatenate([src.reshape(NW, E // NW), t_src], axis=1)
        d_p = jnp.concatenate([dst.reshape(NW, E // NW), t_dst], axis=1)
        return (s_p << 14) | d_p

    pk_g3 = padded(EPW, False).reshape(NW, NCHUNK, CHUNK)
    pk_d3 = padded(EPWD, True).reshape(NW, EPWD // DCHUNK, DCHUNK)

    zero1 = jnp.zeros((NPAD,), jnp.float32)
    zero2 = jnp.zeros((NPAD, H), jnp.float32)

    degp = _deg_call(pk_d3, zero1)
    degp4 = degp.reshape(NC, 2, NPAD, 1)

    b_fc2 = b_fc.reshape(1, H)
    b0_2 = b0.reshape(1, H)
    b1_2 = b1.reshape(1, H)
    b2_2 = b2.reshape(1, C)

    hs0 = _mk_fc()(features_list, W_fc, b_fc2, degp4)
    p1 = _agg_call(hs0, pk_g3, zero2)
    hs1 = _mk_layer_a()(p1, degp4, b0_2)
    p2 = _agg_call(hs1, pk_g3, zero2)
    h2, hs2 = _mk_layer_b()(p2, degp4, W1, b1_2)
    p3 = _agg_call(hs2, pk_g3, zero2)
    out = _mk_layer_c()(p3, degp4, W2, b2_2)
    return (out, h2)


# speedup vs baseline: 1.0705x; 1.0705x over previous
import functools

import jax
import jax.numpy as jnp
from jax import lax
from jax.experimental import pallas as pl
from jax.experimental.pallas import tpu as pltpu
from jax.experimental.pallas import tpu_sc as plsc

N = 10000
E = 320000
D = 128
H = 128
C = 16

NC = 2
NS = 16
NW = NC * NS
CHUNK = 128
NBUF = 2
SPS = 40
NSTAGE = 2
NCHUNK = SPS * NSTAGE
EPW = NCHUNK * CHUNK
EPAD = EPW * NW
DCHUNK = 128
EPWD = 10240
NPAD = 10240
SLAB = NPAD // NS

_MESH = plsc.VectorSubcoreMesh(core_axis_name="c", subcore_axis_name="s")

CPR = 1


def _idx(v, j):
    if CPR == 1:
        return v.at[j]
    return v.at[j // CPR, pl.ds((j % CPR) * CHUNK, CHUNK)]


def _unpack_indices(src_v, dst_v, nchunk, chunk):
    def body(j, carry):
        for k in range(chunk // 16):
            p = src_v[j, pl.ds(k * 16, 16)]
            src_v[j, pl.ds(k * 16, 16)] = lax.shift_right_logical(p, 14)
            dst_v[j, pl.ds(k * 16, 16)] = lax.bitwise_and(p, 16383)
        return carry

    lax.fori_loop(0, nchunk, body, 0)


def _deg_body(pk_hbm, zero_hbm, out_hbm, src_v, dst_v, ones_v,
              dego_s, degi_s):
    c = lax.axis_index("c")
    s = lax.axis_index("s")
    w = c * NS + s
    pltpu.sync_copy(zero_hbm.at[pl.ds(s * SLAB, SLAB)],
                    dego_s.at[pl.ds(s * SLAB, SLAB)])
    pltpu.sync_copy(zero_hbm.at[pl.ds(s * SLAB, SLAB)],
                    degi_s.at[pl.ds(s * SLAB, SLAB)])
    pltpu.sync_copy(pk_hbm.at[w], src_v)
    _unpack_indices(src_v, dst_v, EPWD // DCHUNK, DCHUNK)
    for k in range(DCHUNK // 16):
        ones_v[pl.ds(k * 16, 16)] = jnp.ones((16,), jnp.float32)
    plsc.subcore_barrier()

    def body(j, carry):
        pltpu.sync_copy(ones_v, dego_s.at[src_v.at[j]], add=True)
        pltpu.sync_copy(ones_v, degi_s.at[dst_v.at[j]], add=True)
        return carry

    lax.fori_loop(0, EPWD // DCHUNK, body, 0)
    plsc.subcore_barrier()
    pltpu.sync_copy(dego_s.at[pl.ds(s * SLAB, SLAB)],
                    out_hbm.at[c, 0, pl.ds(s * SLAB, SLAB)])
    pltpu.sync_copy(degi_s.at[pl.ds(s * SLAB, SLAB)],
                    out_hbm.at[c, 1, pl.ds(s * SLAB, SLAB)])


def _agg_body(hs_hbm, pk_hbm, zero_hbm, out_hbm, src_v, dst_v,
              b0, b1, acc_s, g0, g1, s0, s1):
    b2 = g2 = s2 = None
    c = lax.axis_index("c")
    s = lax.axis_index("s")
    w = c * NS + s
    bufs = [b0, b1, b2][:NBUF]
    gsem = [g0, g1, g2][:NBUF]
    ssem = [s0, s1, s2][:NBUF]
    pltpu.sync_copy(zero_hbm.at[pl.ds(s * SLAB, SLAB)],
                    acc_s.at[pl.ds(s * SLAB, SLAB)])
    plsc.subcore_barrier()

    def run_stage(h0):
        pltpu.sync_copy(pk_hbm.at[w, pl.ds(h0, SPS)], src_v)
        _unpack_indices(src_v, dst_v, SPS, CHUNK)

        def g_start(j, b):
            pltpu.async_copy(hs_hbm.at[_idx(src_v, j)], bufs[b], gsem[b])

        def g_wait(j, b):
            pltpu.make_async_copy(hs_hbm.at[_idx(src_v, j)], bufs[b],
                                  gsem[b]).wait()

        def s_start(j, b):
            pltpu.async_copy(bufs[b], acc_s.at[_idx(dst_v, j)], ssem[b],
                             add=True)

        def s_wait(j, b):
            pltpu.make_async_copy(bufs[b], acc_s.at[_idx(dst_v, j)],
                                  ssem[b]).wait()

        g_start(0, 0)

        def body(i, carry):
            for b in range(NBUF):
                j = NBUF * i + b
                g_wait(j, b)
                s_start(j, b)
                nb = (b + 1) % NBUF

                @pl.when(jnp.logical_and(j + 1 < SPS, j >= NBUF - 1))
                def _():
                    s_wait(j - (NBUF - 1), nb)

                @pl.when(j + 1 < SPS)
                def _():
                    g_start(j + 1, nb)

            return carry

        lax.fori_loop(0, SPS // NBUF, body, 0)
        for k in range(NBUF):
            s_wait(SPS - NBUF + k, (SPS - NBUF + k) % NBUF)

    for st in range(NSTAGE):
        run_stage(st * SPS)
    plsc.subcore_barrier()
    pltpu.sync_copy(acc_s.at[pl.ds(s * SLAB, SLAB)],
                    out_hbm.at[c, pl.ds(s * SLAB, SLAB)])


_deg_call = pl.kernel(
    _deg_body,
    out_type=jax.ShapeDtypeStruct((NC, 2, NPAD), jnp.float32),
    mesh=_MESH,
    scratch_types=[
        pltpu.VMEM((EPWD // DCHUNK, DCHUNK), jnp.int32),
        pltpu.VMEM((EPWD // DCHUNK, DCHUNK), jnp.int32),
        pltpu.VMEM((DCHUNK,), jnp.float32),
        pltpu.VMEM_SHARED((NPAD,), jnp.float32),
        pltpu.VMEM_SHARED((NPAD,), jnp.float32),
    ],
)

_agg_call = pl.kernel(
    _agg_body,
    out_type=jax.ShapeDtypeStruct((NC, NPAD, H), jnp.float32),
    mesh=_MESH,
    scratch_types=[
        pltpu.VMEM((SPS, CHUNK), jnp.int32),
        pltpu.VMEM((SPS, CHUNK), jnp.int32),
    ] + [pltpu.VMEM((CHUNK, H), jnp.float32)] * NBUF + [
        pltpu.VMEM_SHARED((NPAD, H), jnp.float32),
    ] + [pltpu.SemaphoreType.DMA] * (2 * NBUF),
)


def _norms(degp):
    normo = lax.rsqrt(jnp.maximum(degp[0, 0] + degp[1, 0], 1.0))
    normi = lax.rsqrt(jnp.maximum(degp[0, 1] + degp[1, 1], 1.0))
    return normo, normi


def _fc_body(x_ref, w_ref, b_ref, degp_ref, out_ref):
    normo, _ = _norms(degp_ref[...])
    h = jnp.dot(x_ref[...], w_ref[...], preferred_element_type=jnp.float32)
    out_ref[...] = (h + b_ref[...]) * normo


def _layer_a_body(p_ref, degp_ref, b_ref, out_ref):
    normo, normi = _norms(degp_ref[...])
    agg = (p_ref[0] + p_ref[1]) * normi
    h = jnp.maximum(agg + b_ref[...], 0.0)
    out_ref[...] = h * normo


def _layer_b_body(p_ref, degp_ref, w_ref, b_ref, h_ref, hs_ref):
    normo, normi = _norms(degp_ref[...])
    agg = (p_ref[0] + p_ref[1]) * normi
    h = jnp.dot(agg, w_ref[...], preferred_element_type=jnp.float32)
    h = jnp.maximum(h + b_ref[...], 0.0)
    h_ref[...] = h
    hs_ref[...] = h * normo


def _layer_c_body(p_ref, degp_ref, w_ref, b_ref, out_ref):
    _, normi = _norms(degp_ref[...])
    agg = (p_ref[0] + p_ref[1]) * normi
    h = jnp.dot(agg, w_ref[...], preferred_element_type=jnp.float32)
    out_ref[...] = h + b_ref[...]


_GRID = (N + 127) // 128

_degp_spec = pl.BlockSpec((NC, 2, 128, 1), lambda i: (0, 0, i, 0))
_row_spec = pl.BlockSpec((128, H), lambda i: (i, 0))
_p_spec = pl.BlockSpec((NC, 128, H), lambda i: (0, i, 0))


def _mk_fc():
    return pl.pallas_call(
        _fc_body,
        grid=(_GRID,),
        in_specs=[_row_spec,
                  pl.BlockSpec((D, H), lambda i: (0, 0)),
                  pl.BlockSpec((1, H), lambda i: (0, 0)),
                  _degp_spec],
        out_specs=_row_spec,
        out_shape=jax.ShapeDtypeStruct((N, H), jnp.float32),
    )


def _mk_layer_a():
    return pl.pallas_call(
        _layer_a_body,
        grid=(_GRID,),
        in_specs=[_p_spec, _degp_spec,
                  pl.BlockSpec((1, H), lambda i: (0, 0))],
        out_specs=_row_spec,
        out_shape=jax.ShapeDtypeStruct((N, H), jnp.float32),
    )


def _mk_layer_b():
    return pl.pallas_call(
        _layer_b_body,
        grid=(_GRID,),
        in_specs=[_p_spec, _degp_spec,
                  pl.BlockSpec((H, H), lambda i: (0, 0)),
                  pl.BlockSpec((1, H), lambda i: (0, 0))],
        out_specs=[_row_spec, _row_spec],
        out_shape=[jax.ShapeDtypeStruct((N, H), jnp.float32),
                   jax.ShapeDtypeStruct((N, H), jnp.float32)],
    )


def _mk_layer_c():
    return pl.pallas_call(
        _layer_c_body,
        grid=(_GRID,),
        in_specs=[_p_spec, _degp_spec,
                  pl.BlockSpec((H, C), lambda i: (0, 0)),
                  pl.BlockSpec((1, C), lambda i: (0, 0))],
        out_specs=pl.BlockSpec((128, C), lambda i: (i, 0)),
        out_shape=jax.ShapeDtypeStruct((N, C), jnp.float32),
    )


def kernel(features_list, edge_index, e_feat, W_fc, b_fc, b0, W1, b1, W2, b2):
    src = edge_index[0]
    dst = edge_index[1]
    def padded(epw, trash_src):
        padw = epw - E // NW
        tr = jnp.arange(padw, dtype=jnp.int32)
        t_dst = jnp.broadcast_to(N + tr % (NPAD - N), (NW, padw))
        t_src = t_dst if trash_src else jnp.broadcast_to(tr % N, (NW, padw))
        s_p = jnp.concatenate([src.reshape(NW, E // NW), t_src], axis=1)
        d_p = jnp.concatenate([dst.reshape(NW, E // NW), t_dst], axis=1)
        return (s_p << 14) | d_p

    pk_g3 = padded(EPW, False).reshape(NW, NCHUNK, CHUNK)
    pk_d3 = padded(EPWD, True).reshape(NW, EPWD // DCHUNK, DCHUNK)

    zero1 = jnp.zeros((NPAD,), jnp.float32)
    zero2 = jnp.zeros((NPAD, H), jnp.float32)

    degp = _deg_call(pk_d3, zero1)
    degp4 = degp.reshape(NC, 2, NPAD, 1)

    b_fc2 = b_fc.reshape(1, H)
    b0_2 = b0.reshape(1, H)
    b1_2 = b1.reshape(1, H)
    b2_2 = b2.reshape(1, C)

    hs0 = _mk_fc()(features_list, W_fc, b_fc2, degp4)
    p1 = _agg_call(hs0, pk_g3, zero2)
    hs1 = _mk_layer_a()(p1, degp4, b0_2)
    p2 = _agg_call(hs1, pk_g3, zero2)
    h2, hs2 = _mk_layer_b()(p2, degp4, W1, b1_2)
    p3 = _agg_call(hs2, pk_g3, zero2)
    out = _mk_layer_c()(p3, degp4, W2, b2_2)
    return (out, h2)
